# Initial kernel scaffold; baseline (speedup 1.0000x reference)
import jax
import jax.numpy as jnp
from jax.experimental import pallas as pl


def kernel(inputs, codebook, ema_cs_hidden, ema_dw_hidden, counter, training):
    raise NotImplementedError("write your pallas kernel here")


# speedup vs baseline: 1.5964x; 1.5964x over previous
import functools

import jax
import jax.numpy as jnp
from jax.experimental import pallas as pl
from jax.experimental.pallas import tpu as pltpu
from jax.experimental.pallas import tpu_sc as plsc

NUM_EMBEDDINGS = 512
EMBEDDING_DIM = 32
BETA = 0.25
ROW_TILE = 2048
GATHER_WINDOW = 128


def _distance_body(z_ref, cb_ref, cn_ref, idx_ref, counts_ref, dsum_ref,
                   q_ref):
    z = z_ref[...]
    cb = cb_ref[...]
    dot = jnp.dot(z, cb, preferred_element_type=jnp.float32)
    znorm = jnp.sum(z * z, axis=1, keepdims=True)
    d = znorm - 2.0 * dot + cn_ref[...]
    dmin = jnp.min(d, axis=1, keepdims=True)
    k_iota = jax.lax.broadcasted_iota(jnp.int32, d.shape, 1)
    idx = jnp.min(jnp.where(d == dmin, k_iota, NUM_EMBEDDINGS), axis=1)
    idx_ref[0, 0, :] = idx
    onehot = (k_iota == idx[:, None]).astype(jnp.float32)
    counts_ref[0, 0, :] = jnp.sum(onehot, axis=0)
    dsum_ref[0, 0, :] = jnp.full((128,), jnp.sum(dmin), jnp.float32)
    q_ref[...] = jax.lax.dot_general(
        onehot, cb, (((1,), (1,)), ((), ())),
        precision=jax.lax.Precision.HIGHEST,
        preferred_element_type=jnp.float32)


def _finalize_body(counts_ref, dsum_ref, perp_ref, loss_ref, n_rows):
    counts = jnp.sum(counts_ref[...], axis=(0, 1))
    avg = counts * (1.0 / n_rows)
    perp = jnp.exp(-jnp.sum(avg * jnp.log(avg + 1e-10)))
    perp_ref[...] = jnp.full((1, 1), perp, jnp.float32)
    total = jnp.sum(dsum_ref[:, :, 0])
    loss = total * (BETA / (n_rows * EMBEDDING_DIM))
    loss_ref[...] = jnp.full((1, 1), loss, jnp.float32)


GATHER_CHUNK = 128
ROW_BYTES = EMBEDDING_DIM * 4


def _sc_gather(table_u8, idx2, n_rows):
    info = plsc.get_sparse_core_info()
    n_workers = info.num_cores * info.num_subcores
    chunks_per_w = n_rows // (n_workers * GATHER_CHUNK)
    mesh = plsc.VectorSubcoreMesh(core_axis_name="c", subcore_axis_name="s")

    @functools.partial(
        pl.kernel,
        mesh=mesh,
        out_type=jax.ShapeDtypeStruct((n_rows, ROW_BYTES), jnp.uint8),
        scratch_types=[
            pltpu.VMEM((chunks_per_w, GATHER_CHUNK), jnp.int32),
            pltpu.VMEM((GATHER_CHUNK, ROW_BYTES), jnp.uint8),
            pltpu.VMEM((GATHER_CHUNK, ROW_BYTES), jnp.uint8),
            pltpu.SemaphoreType.DMA,
            pltpu.SemaphoreType.DMA,
        ],
    )
    def gather_kernel(table_hbm, idx_hbm, out_hbm, idx_v, rows0, rows1, s0, s1):
        wid = jax.lax.axis_index("s") * info.num_cores + jax.lax.axis_index("c")
        cbase = wid * chunks_per_w
        pltpu.sync_copy(idx_hbm.at[pl.ds(cbase, chunks_per_w)], idx_v)
        rows = (rows0, rows1)
        sems = (s0, s1)

        @pl.loop(0, chunks_per_w, step=2)
        def _(j):
            for b in range(2):
                pltpu.async_copy(table_hbm.at[idx_v.at[j + b]], rows[b],
                                 sems[b]).wait()
                row0 = (cbase + j + b) * GATHER_CHUNK
                pltpu.sync_copy(rows[b], out_hbm.at[pl.ds(row0, GATHER_CHUNK)])

    return gather_kernel(table_u8, idx2)


def kernel(inputs, codebook, ema_cs_hidden, ema_dw_hidden, counter, training):
    batch, hw, dim = inputs.shape
    n_rows = batch * hw
    z = inputs.reshape(n_rows, dim)
    n_tiles = n_rows // ROW_TILE
    cnorm = jnp.sum(codebook * codebook, axis=0, keepdims=True)

    idx3, counts3, dsum3, q = pl.pallas_call(
        _distance_body,
        grid=(n_tiles,),
        in_specs=[
            pl.BlockSpec((ROW_TILE, dim), lambda i: (i, 0)),
            pl.BlockSpec((dim, NUM_EMBEDDINGS), lambda i: (0, 0)),
            pl.BlockSpec((1, NUM_EMBEDDINGS), lambda i: (0, 0)),
        ],
        out_specs=[
            pl.BlockSpec((1, 1, ROW_TILE), lambda i: (i, 0, 0)),
            pl.BlockSpec((1, 1, NUM_EMBEDDINGS), lambda i: (i, 0, 0)),
            pl.BlockSpec((1, 1, 128), lambda i: (i, 0, 0)),
            pl.BlockSpec((ROW_TILE, dim), lambda i: (i, 0)),
        ],
        out_shape=[
            jax.ShapeDtypeStruct((n_tiles, 1, ROW_TILE), jnp.int32),
            jax.ShapeDtypeStruct((n_tiles, 1, NUM_EMBEDDINGS), jnp.float32),
            jax.ShapeDtypeStruct((n_tiles, 1, 128), jnp.float32),
            jax.ShapeDtypeStruct((n_rows, dim), jnp.float32),
        ],
        compiler_params=pltpu.CompilerParams(
            dimension_semantics=("parallel",),
        ),
    )(z, codebook, cnorm)

    perp2, loss2 = pl.pallas_call(
        lambda c_ref, d_ref, p_ref, l_ref: _finalize_body(
            c_ref, d_ref, p_ref, l_ref, n_rows),
        in_specs=[
            pl.BlockSpec((n_tiles, 1, NUM_EMBEDDINGS), lambda: (0, 0, 0)),
            pl.BlockSpec((n_tiles, 1, 128), lambda: (0, 0, 0)),
        ],
        out_specs=[
            pl.BlockSpec((1, 1), lambda: (0, 0)),
            pl.BlockSpec((1, 1), lambda: (0, 0)),
        ],
        out_shape=[
            jax.ShapeDtypeStruct((1, 1), jnp.float32),
            jax.ShapeDtypeStruct((1, 1), jnp.float32),
        ],
    )(counts3, dsum3)

    ste = q.reshape(batch, hw, dim)
    return ste, perp2.reshape(()), loss2.reshape(())


# speedup vs baseline: 3.7331x; 2.3384x over previous
import functools

import jax
import jax.numpy as jnp
from jax.experimental import pallas as pl
from jax.experimental.pallas import tpu as pltpu
from jax.experimental.pallas import tpu_sc as plsc

NUM_EMBEDDINGS = 512
EMBEDDING_DIM = 32
BETA = 0.25
ROW_TILE = 2048
GATHER_WINDOW = 128


def _distance_body(z_ref, cb_ref, cn_ref, counts_ref, dsum_ref, q_ref):
    z = z_ref[...]
    cb = cb_ref[...]
    dot2 = jnp.dot(z + z, cb, preferred_element_type=jnp.float32)
    znorm = jnp.sum(z * z, axis=1, keepdims=True)
    d = znorm - dot2 + cn_ref[...]
    dmin = jnp.min(d, axis=1, keepdims=True)
    k_iota = jax.lax.broadcasted_iota(jnp.int32, d.shape, 1).astype(jnp.float32)
    idxf = jnp.min(jnp.where(d == dmin, k_iota, float(NUM_EMBEDDINGS)),
                   axis=1, keepdims=True)
    onehot = (k_iota == idxf).astype(jnp.float32)
    counts_ref[0, 0, :] = jnp.sum(onehot, axis=0)
    dsum_ref[0, 0, :] = jnp.full((128,), jnp.sum(dmin), jnp.float32)
    q_ref[...] = jax.lax.dot_general(
        onehot, cb, (((1,), (1,)), ((), ())),
        preferred_element_type=jnp.float32)


def _finalize_body(counts_ref, dsum_ref, perp_ref, loss_ref, n_rows):
    counts = jnp.sum(counts_ref[...], axis=(0, 1))
    avg = counts * (1.0 / n_rows)
    perp = jnp.exp(-jnp.sum(avg * jnp.log(avg + 1e-10)))
    perp_ref[...] = jnp.full((1, 1), perp, jnp.float32)
    total = jnp.sum(dsum_ref[:, :, 0])
    loss = total * (BETA / (n_rows * EMBEDDING_DIM))
    loss_ref[...] = jnp.full((1, 1), loss, jnp.float32)


GATHER_CHUNK = 128
ROW_BYTES = EMBEDDING_DIM * 4


def _sc_gather(table_u8, idx2, n_rows):
    info = plsc.get_sparse_core_info()
    n_workers = info.num_cores * info.num_subcores
    chunks_per_w = n_rows // (n_workers * GATHER_CHUNK)
    mesh = plsc.VectorSubcoreMesh(core_axis_name="c", subcore_axis_name="s")

    @functools.partial(
        pl.kernel,
        mesh=mesh,
        out_type=jax.ShapeDtypeStruct((n_rows, ROW_BYTES), jnp.uint8),
        scratch_types=[
            pltpu.VMEM((chunks_per_w, GATHER_CHUNK), jnp.int32),
            pltpu.VMEM((GATHER_CHUNK, ROW_BYTES), jnp.uint8),
            pltpu.VMEM((GATHER_CHUNK, ROW_BYTES), jnp.uint8),
            pltpu.SemaphoreType.DMA,
            pltpu.SemaphoreType.DMA,
        ],
    )
    def gather_kernel(table_hbm, idx_hbm, out_hbm, idx_v, rows0, rows1, s0, s1):
        wid = jax.lax.axis_index("s") * info.num_cores + jax.lax.axis_index("c")
        cbase = wid * chunks_per_w
        pltpu.sync_copy(idx_hbm.at[pl.ds(cbase, chunks_per_w)], idx_v)
        rows = (rows0, rows1)
        sems = (s0, s1)

        @pl.loop(0, chunks_per_w, step=2)
        def _(j):
            for b in range(2):
                pltpu.async_copy(table_hbm.at[idx_v.at[j + b]], rows[b],
                                 sems[b]).wait()
                row0 = (cbase + j + b) * GATHER_CHUNK
                pltpu.sync_copy(rows[b], out_hbm.at[pl.ds(row0, GATHER_CHUNK)])

    return gather_kernel(table_u8, idx2)


def kernel(inputs, codebook, ema_cs_hidden, ema_dw_hidden, counter, training):
    batch, hw, dim = inputs.shape
    n_rows = batch * hw
    z = inputs.reshape(n_rows, dim)
    n_tiles = n_rows // ROW_TILE
    cnorm = jnp.sum(codebook * codebook, axis=0, keepdims=True)

    counts3, dsum3, q = pl.pallas_call(
        _distance_body,
        grid=(n_tiles,),
        in_specs=[
            pl.BlockSpec((ROW_TILE, dim), lambda i: (i, 0)),
            pl.BlockSpec((dim, NUM_EMBEDDINGS), lambda i: (0, 0)),
            pl.BlockSpec((1, NUM_EMBEDDINGS), lambda i: (0, 0)),
        ],
        out_specs=[
            pl.BlockSpec((1, 1, NUM_EMBEDDINGS), lambda i: (i, 0, 0)),
            pl.BlockSpec((1, 1, 128), lambda i: (i, 0, 0)),
            pl.BlockSpec((ROW_TILE, dim), lambda i: (i, 0)),
        ],
        out_shape=[
            jax.ShapeDtypeStruct((n_tiles, 1, NUM_EMBEDDINGS), jnp.float32),
            jax.ShapeDtypeStruct((n_tiles, 1, 128), jnp.float32),
            jax.ShapeDtypeStruct((n_rows, dim), jnp.float32),
        ],
        compiler_params=pltpu.CompilerParams(
            dimension_semantics=("parallel",),
        ),
    )(z, codebook, cnorm)

    perp2, loss2 = pl.pallas_call(
        lambda c_ref, d_ref, p_ref, l_ref: _finalize_body(
            c_ref, d_ref, p_ref, l_ref, n_rows),
        in_specs=[
            pl.BlockSpec((n_tiles, 1, NUM_EMBEDDINGS), lambda: (0, 0, 0)),
            pl.BlockSpec((n_tiles, 1, 128), lambda: (0, 0, 0)),
        ],
        out_specs=[
            pl.BlockSpec((1, 1), lambda: (0, 0)),
            pl.BlockSpec((1, 1), lambda: (0, 0)),
        ],
        out_shape=[
            jax.ShapeDtypeStruct((1, 1), jnp.float32),
            jax.ShapeDtypeStruct((1, 1), jnp.float32),
        ],
    )(counts3, dsum3)

    ste = q.reshape(batch, hw, dim)
    return ste, perp2.reshape(()), loss2.reshape(())
